# baseline (speedup 1.0000x reference)
import functools

import jax
import jax.numpy as jnp
from jax import lax
from jax.experimental import pallas as pl
from jax.experimental.pallas import tpu as pltpu
from jax.experimental.pallas import tpu_sc as plsc

_S = 25
_SP = 32
_R = 256
_POS = 1e30
_G2 = 16
_CAP = 496

_W_CACHE = {}


def _w_const(b, n):
    if (b, n) not in _W_CACHE:
        u = jax.random.uniform(jax.random.key(42), (b * n * n,),
                               dtype=jnp.float32, minval=1e-10, maxval=1.0)
        _W_CACHE[(b, n)] = 1.0 / (-jnp.log(u))
    return _W_CACHE[(b, n)]


def _stage1_body(d_ref, w_ref, idx_ref, *, rows_per_blk, n):
    row_base = pl.program_id(1) * rows_per_blk
    d = d_ref[0]
    w = w_ref[0]
    col_ids = jax.lax.broadcasted_iota(jnp.int32, (rows_per_blk, n), 1)
    row_ids = row_base + jax.lax.broadcasted_iota(
        jnp.int32, (rows_per_blk, n), 0)
    dproc = jnp.where(col_ids == row_ids, 1e6, d)
    s = w / (dproc + 1e-6)
    lane = jax.lax.broadcasted_iota(jnp.int32, (rows_per_blk, _SP), 1)

    def topk_body(k, carry):
        s, iacc = carry
        m = jnp.max(s, axis=1, keepdims=True)
        cand = jnp.where(s >= m, col_ids, n)
        jm = jnp.min(cand, axis=1, keepdims=True)
        iacc = jnp.where(lane == k, jm, iacc)
        s = jnp.where(cand == jm, -1.0, s)
        return s, iacc

    iacc0 = jnp.zeros((rows_per_blk, _SP), jnp.int32)
    _, iacc = jax.lax.fori_loop(0, _S, topk_body, (s, iacc0))
    idx_ref[0] = iacc


def _sc_gather_body(dist_hbm, idx_hbm, rows_hbm, cols_hbm,
                    idxblk, flat, vals, outbuf, sem, *, bs, n, nc, nw, b_off):
    rows_per_w = (bs * n) // nw
    n_groups = rows_per_w // _G2
    wid = lax.axis_index("s") * nc + lax.axis_index("c")
    lane16 = lax.iota(jnp.int32, 16)
    padmask = lane16 >= (_S - 16)

    def vsortf(x):
        return plsc.sort_key_val(x, x)[0]

    def sort25(v0, v1):
        v1 = jnp.where(padmask, _POS, v1)
        a = vsortf(v0)
        c = vsortf(v1)
        rc = lax.rev(c, (0,))
        lo = vsortf(jnp.minimum(a, rc))
        hi = vsortf(jnp.maximum(a, rc))
        hi = jnp.where(padmask, 0.0, hi)
        return lo, hi

    def group_body(g, _):
        r0 = wid * rows_per_w + g * _G2
        pltpu.sync_copy(idx_hbm.at[pl.ds(r0 * _SP, _G2 * _SP)], idxblk)

        def build_body(r, _):
            rg = r0 + r
            bb = rg // n
            ii = rg - bb * n
            rowbase = (b_off + bb) * n * n + ii * n
            colbase = (b_off + bb) * n * n + ii
            j0 = idxblk[pl.ds(r * _SP, 16)]
            j1 = idxblk[pl.ds(r * _SP + 16, 16)]
            flat[pl.ds(r * 64, 16)] = rowbase + j0
            flat[pl.ds(r * 64 + 16, 16)] = rowbase + j1
            flat[pl.ds(r * 64 + 32, 16)] = j0 * n + colbase
            flat[pl.ds(r * 64 + 48, 16)] = j1 * n + colbase
            return 0

        lax.fori_loop(0, _G2, build_body, 0)
        copies = [
            pltpu.async_copy(dist_hbm.at[flat.at[pl.ds(c * 128, 128)]],
                             vals.at[pl.ds(c * 128, 128)], sem)
            for c in range(_G2 * 64 // 128)
        ]
        for cp in copies:
            cp.wait()

        def sort_body(r, _):
            rlo, rhi = sort25(vals[pl.ds(r * 64, 16)],
                              vals[pl.ds(r * 64 + 16, 16)])
            clo, chi = sort25(vals[pl.ds(r * 64 + 32, 16)],
                              vals[pl.ds(r * 64 + 48, 16)])
            outbuf[pl.ds(r * _SP, 16)] = rlo
            outbuf[pl.ds(r * _SP + 16, 16)] = rhi
            outbuf[pl.ds(_G2 * _SP + r * _SP, 16)] = clo
            outbuf[pl.ds(_G2 * _SP + r * _SP + 16, 16)] = chi
            return 0

        lax.fori_loop(0, _G2, sort_body, 0)
        pltpu.sync_copy(outbuf.at[pl.ds(0, _G2 * _SP)],
                        rows_hbm.at[pl.ds(r0 * _SP, _G2 * _SP)])
        pltpu.sync_copy(outbuf.at[pl.ds(_G2 * _SP, _G2 * _SP)],
                        cols_hbm.at[pl.ds(r0 * _SP, _G2 * _SP)])
        return 0

    lax.fori_loop(0, n_groups, group_body, 0)


def _sc_gather_sort(dist_flat, idx_flat, bs, n, b_off):
    info = plsc.get_sparse_core_info()
    nc, ns = info.num_cores, info.num_subcores
    nw = nc * ns
    mesh = plsc.VectorSubcoreMesh(core_axis_name="c", subcore_axis_name="s")
    kern = functools.partial(
        pl.kernel,
        mesh=mesh,
        compiler_params=pltpu.CompilerParams(needs_layout_passes=False),
        out_type=[jax.ShapeDtypeStruct((bs * n * _SP,), jnp.float32)] * 2,
        scratch_types=[
            pltpu.VMEM((_G2 * _SP,), jnp.int32),
            pltpu.VMEM((_G2 * 64,), jnp.int32),
            pltpu.VMEM((_G2 * 64,), jnp.float32),
            pltpu.VMEM((2 * _G2 * _SP,), jnp.float32),
            pltpu.SemaphoreType.DMA,
        ],
    )(functools.partial(_sc_gather_body, bs=bs, n=n, nc=nc, nw=nw,
                        b_off=b_off))
    return kern(dist_flat, idx_flat)


def _sc_body(dist_hbm, w_hbm, rows_hbm, cols_hbm,
             dbuf, wbuf, qbuf, candq, candj, colidx, colvals, outbuf, sem,
             *, b, n, nc, nw):
    rows_per_w = (b * n) // nw
    n_groups = rows_per_w // _G2
    wid = lax.axis_index("s") * nc + lax.axis_index("c")
    lane16 = lax.iota(jnp.int32, 16)
    padmask = lane16 >= (_S - 16)
    nchunks = n // 16

    def vsortf(x):
        return plsc.sort_key_val(x, x)[0]

    def sort25(v0, v1):
        v1 = jnp.where(padmask, _POS, v1)
        a = vsortf(v0)
        c = vsortf(v1)
        rc = lax.rev(c, (0,))
        lo = vsortf(jnp.minimum(a, rc))
        hi = vsortf(jnp.maximum(a, rc))
        hi = jnp.where(padmask, 0.0, hi)
        return lo, hi

    def group_body(g, _):
        rg0 = wid * rows_per_w + g * _G2
        bb = rg0 // n
        i0 = rg0 - bb * n
        src0 = rg0 * n
        pltpu.sync_copy(dist_hbm.at[pl.ds(src0, _G2 * n)], dbuf)
        pltpu.sync_copy(w_hbm.at[pl.ds(src0, _G2 * n)], wbuf)

        def row_body(r, _):
            i_row = i0 + r
            roff = r * n

            def scan1(c, carry):
                m1, m2 = carry
                d = dbuf[pl.ds(roff + c * 16, 16)]
                wv = wbuf[pl.ds(roff + c * 16, 16)]
                jg = c * 16 + lane16
                dp = jnp.where(jg == i_row, 1e6, d)
                q = wv / (dp + 1e-6)
                qb = plsc.bitcast(q, jnp.int32)
                qbuf[pl.ds(c * 16, 16)] = qb
                gt1 = qb > m1
                gt2 = qb > m2
                m2 = jnp.where(gt1, m1, jnp.where(gt2, qb, m2))
                m1 = jnp.where(gt1, qb, m1)
                return m1, m2

            minit = jnp.full((16,), -1, jnp.int32)
            _, m2 = lax.fori_loop(0, nchunks, scan1, (minit, minit))
            t0 = jnp.min(m2)

            def scan2(c, base):
                qb = qbuf[pl.ds(c * 16, 16)]
                m = qb >= t0
                cs = plsc.cumsum(jnp.where(m, 1, 0))
                pos = base + cs - 1
                mm = m & (pos < _CAP)
                plsc.store_scatter(candq, [pos], qb, mask=mm)
                plsc.store_scatter(candj, [pos], c * 16 + lane16, mask=mm)
                return base + plsc.all_reduce_population_count(m)

            base = lax.fori_loop(0, nchunks, scan2,
                                 jnp.zeros((16,), jnp.int32))
            cnt = jnp.max(base)
            nv = (cnt + 15) // 16

            def merge(v, carry):
                u, ui, vv, vi = carry
                ck = candq[pl.ds(v * 16, 16)]
                cj = candj[pl.ds(v * 16, 16)]
                valid = (v * 16 + lane16) < cnt
                ck = jnp.where(valid, ck, -1)
                ks, js = plsc.sort_key_val(ck, cj, descending=True)
                rk = lax.rev(ks, (0,))
                rj = lax.rev(js, (0,))
                m = u >= rk
                u2, ui2 = plsc.sort_key_val(
                    jnp.where(m, u, rk), jnp.where(m, ui, rj),
                    descending=True)
                sk, sv = plsc.sort_key_val(
                    jnp.where(m, rk, u), jnp.where(m, rj, ui),
                    descending=True)
                rsk = lax.rev(sk, (0,))
                rsv = lax.rev(sv, (0,))
                m2_ = vv >= rsk
                v2, vi2 = plsc.sort_key_val(
                    jnp.where(m2_, vv, rsk), jnp.where(m2_, vi, rsv),
                    descending=True)
                return u2, ui2, v2, vi2

            mneg = jnp.full((16,), -1, jnp.int32)
            zer = jnp.zeros((16,), jnp.int32)
            _, ui, _, vi = lax.fori_loop(0, nv, merge,
                                         (mneg, zer, mneg, zer))

            rv0 = plsc.load_gather(dbuf, [roff + ui])
            rv1 = plsc.load_gather(dbuf, [roff + vi])
            rlo, rhi = sort25(rv0, rv1)
            outbuf[pl.ds(r * _SP, 16)] = rlo
            outbuf[pl.ds(r * _SP + 16, 16)] = rhi
            colbase = bb * n * n + i_row
            colidx[pl.ds(r * _SP, 16)] = ui * n + colbase
            colidx[pl.ds(r * _SP + 16, 16)] = vi * n + colbase
            return 0

        lax.fori_loop(0, _G2, row_body, 0)

        copies = [
            pltpu.async_copy(dist_hbm.at[colidx.at[pl.ds(c * 128, 128)]],
                             colvals.at[pl.ds(c * 128, 128)], sem)
            for c in range(_G2 * _SP // 128)
        ]
        for cp in copies:
            cp.wait()

        def col_sort_body(r, _):
            clo, chi = sort25(colvals[pl.ds(r * _SP, 16)],
                              colvals[pl.ds(r * _SP + 16, 16)])
            outbuf[pl.ds(_G2 * _SP + r * _SP, 16)] = clo
            outbuf[pl.ds(_G2 * _SP + r * _SP + 16, 16)] = chi
            return 0

        lax.fori_loop(0, _G2, col_sort_body, 0)
        dst0 = rg0 * _SP
        pltpu.sync_copy(outbuf.at[pl.ds(0, _G2 * _SP)],
                        rows_hbm.at[pl.ds(dst0, _G2 * _SP)])
        pltpu.sync_copy(outbuf.at[pl.ds(_G2 * _SP, _G2 * _SP)],
                        cols_hbm.at[pl.ds(dst0, _G2 * _SP)])
        return 0

    lax.fori_loop(0, n_groups, group_body, 0)


def _sc_sample_gather_sort(dist_flat, w_flat, b, n):
    info = plsc.get_sparse_core_info()
    nc, ns = info.num_cores, info.num_subcores
    nw = nc * ns
    mesh = plsc.VectorSubcoreMesh(core_axis_name="c", subcore_axis_name="s")
    kern = functools.partial(
        pl.kernel,
        mesh=mesh,
        compiler_params=pltpu.CompilerParams(needs_layout_passes=False),
        out_type=[jax.ShapeDtypeStruct((b * n * _SP,), jnp.float32)] * 2,
        scratch_types=[
            pltpu.VMEM((_G2 * n,), jnp.float32),
            pltpu.VMEM((_G2 * n,), jnp.float32),
            pltpu.VMEM((n,), jnp.int32),
            pltpu.VMEM((_CAP + 16,), jnp.int32),
            pltpu.VMEM((_CAP + 16,), jnp.int32),
            pltpu.VMEM((_G2 * _SP,), jnp.int32),
            pltpu.VMEM((_G2 * _SP,), jnp.float32),
            pltpu.VMEM((2 * _G2 * _SP,), jnp.float32),
            pltpu.SemaphoreType.DMA,
        ],
    )(functools.partial(_sc_body, b=b, n=n, nc=nc, nw=nw))
    return kern(dist_flat, w_flat)


def _stage2_body(locs_ref, rows_ref, cols_ref, iwt_ref, rwt_ref, cwt_ref,
                 g1c_r_ref, g1d_r_ref, g1c_c_ref, g1d_c_ref, aux_ref,
                 b128_ref, outr_ref, outc_ref):
    f32 = jnp.float32
    aux = aux_ref[...]
    b128 = b128_ref[...]
    e = (jnp.dot(locs_ref[0], iwt_ref[...], preferred_element_type=f32)
         + b128[0:1, :])
    remb = (jnp.dot(rows_ref[0], rwt_ref[...], preferred_element_type=f32)
            + b128[1:2, :])
    cemb = (jnp.dot(cols_ref[0], cwt_ref[...], preferred_element_type=f32)
            + b128[2:3, :])

    def gate(feat, w1c, w1d, brow, wrow, b2row):
        h = jax.nn.relu(
            jnp.dot(e, w1c, preferred_element_type=f32)
            + jnp.dot(feat, w1d, preferred_element_type=f32)
            + aux[brow:brow + 1, :])
        gp = (jnp.sum(h * aux[wrow:wrow + 1, :], axis=1, keepdims=True)
              + aux[b2row:b2row + 1, 0:1])
        g = jax.nn.sigmoid(gp)
        return g * e + (1.0 - g) * feat

    outr_ref[0] = gate(remb, g1c_r_ref[...], g1d_r_ref[...], 0, 2, 4)
    outc_ref[0] = gate(cemb, g1c_c_ref[...], g1d_c_ref[...], 1, 3, 5)


def kernel(locs, distance_matrix, params):
    b, n, _ = locs.shape
    f32 = jnp.float32

    w = _w_const(b, n)
    dist_flat = distance_matrix.reshape(-1)

    bs_sc = b // 2
    bs_tc = b - bs_sc
    sc_rows_f, sc_cols_f = _sc_sample_gather_sort(dist_flat, w, bs_sc, n)

    w_tc = w[bs_sc * n * n:].reshape(bs_tc, n, n)
    rows_per_blk = _R if n % _R == 0 else n
    big_spec = pl.BlockSpec((1, rows_per_blk, n),
                            lambda i, j: (i + bs_sc, j, 0))
    w_spec = pl.BlockSpec((1, rows_per_blk, n), lambda i, j: (i, j, 0))
    idx = pl.pallas_call(
        functools.partial(_stage1_body, rows_per_blk=rows_per_blk, n=n),
        grid=(bs_tc, n // rows_per_blk),
        in_specs=[big_spec, w_spec],
        out_specs=pl.BlockSpec((1, rows_per_blk, _SP), lambda i, j: (i, j, 0)),
        out_shape=jax.ShapeDtypeStruct((bs_tc, n, _SP), jnp.int32),
    )(distance_matrix, w_tc)

    tc_rows_f, tc_cols_f = _sc_gather_sort(
        dist_flat, idx.reshape(-1), bs_tc, n, bs_sc)

    rows_sorted = jnp.concatenate(
        [sc_rows_f.reshape(bs_sc, n, _SP), tc_rows_f.reshape(bs_tc, n, _SP)],
        axis=0)
    cols_sorted = jnp.concatenate(
        [sc_cols_f.reshape(bs_sc, n, _SP), tc_cols_f.reshape(bs_tc, n, _SP)],
        axis=0)

    locs_pad = jnp.pad(locs, ((0, 0), (0, 0), (0, 6)))
    iwt = jnp.pad(params['init_W'].T, ((0, 6), (0, 0)))
    rwt = jnp.pad(params['row_W'].T, ((0, _SP - _S), (0, 0)))
    cwt = jnp.pad(params['col_W'].T, ((0, _SP - _S), (0, 0)))
    g1_r = params['grow_W1'].T
    g1_c = params['gcol_W1'].T
    ed = g1_r.shape[0] // 2
    aux = jnp.zeros((8, 2 * ed), f32)
    aux = aux.at[0, :].set(params['grow_b1'])
    aux = aux.at[1, :].set(params['gcol_b1'])
    aux = aux.at[2, :].set(params['grow_W2'][0])
    aux = aux.at[3, :].set(params['gcol_W2'][0])
    aux = aux.at[4, :].set(params['grow_b2'][0])
    aux = aux.at[5, :].set(params['gcol_b2'][0])
    b128 = jnp.zeros((8, ed), f32)
    b128 = b128.at[0, :].set(params['init_b'])
    b128 = b128.at[1, :].set(params['row_b'])
    b128 = b128.at[2, :].set(params['col_b'])

    def wspec(shape):
        return pl.BlockSpec(shape, lambda i: (0,) * len(shape))

    outr, outc = pl.pallas_call(
        _stage2_body,
        grid=(b,),
        in_specs=[
            pl.BlockSpec((1, n, 8), lambda i: (i, 0, 0)),
            pl.BlockSpec((1, n, _SP), lambda i: (i, 0, 0)),
            pl.BlockSpec((1, n, _SP), lambda i: (i, 0, 0)),
            wspec((8, ed)), wspec((_SP, ed)), wspec((_SP, ed)),
            wspec((ed, 2 * ed)), wspec((ed, 2 * ed)),
            wspec((ed, 2 * ed)), wspec((ed, 2 * ed)),
            wspec((8, 2 * ed)), wspec((8, ed)),
        ],
        out_specs=[pl.BlockSpec((1, n, ed), lambda i: (i, 0, 0))] * 2,
        out_shape=[jax.ShapeDtypeStruct((b, n, ed), f32)] * 2,
    )(locs_pad, rows_sorted, cols_sorted, iwt, rwt, cwt,
      g1_r[:ed], g1_r[ed:], g1_c[:ed], g1_c[ed:], aux, b128)

    return (outr, outc, distance_matrix)


# speedup vs baseline: 1.2115x; 1.2115x over previous
import functools

import jax
import jax.numpy as jnp
from jax import lax
from jax.experimental import pallas as pl
from jax.experimental.pallas import tpu as pltpu
from jax.experimental.pallas import tpu_sc as plsc

_S = 25
_SP = 32
_R = 256
_NEG = -1.0
_POS = 1e30
_G = 64

_W_CACHE = {}


def _w_const(b, n):
    if (b, n) not in _W_CACHE:
        u = jax.random.uniform(jax.random.key(42), (b, n, n),
                               dtype=jnp.float32, minval=1e-10, maxval=1.0)
        _W_CACHE[(b, n)] = 1.0 / (-jnp.log(u))
    return _W_CACHE[(b, n)]


def _stage1_body(d_ref, w_ref, idx_ref, *, rows_per_blk, n):
    row_base = pl.program_id(1) * rows_per_blk
    d = d_ref[0]
    w = w_ref[0]
    col_ids = jax.lax.broadcasted_iota(jnp.int32, (rows_per_blk, n), 1)
    row_ids = row_base + jax.lax.broadcasted_iota(
        jnp.int32, (rows_per_blk, n), 0)
    dproc = jnp.where(col_ids == row_ids, 1e6, d)
    s = w / (dproc + 1e-6)

    lane = jax.lax.broadcasted_iota(jnp.int32, (rows_per_blk, _SP), 1)

    def topk_body(k, carry):
        s, iacc = carry
        m = jnp.max(s, axis=1, keepdims=True)
        cand = jnp.where(s >= m, col_ids, n)
        jm = jnp.min(cand, axis=1, keepdims=True)
        iacc = jnp.where(lane == k, jm, iacc)
        s = jnp.where(cand == jm, _NEG, s)
        return s, iacc

    iacc0 = jnp.zeros((rows_per_blk, _SP), jnp.int32)
    _, iacc = jax.lax.fori_loop(0, _S, topk_body, (s, iacc0))
    idx_ref[0] = iacc


def _sc_body(dist_hbm, idx_hbm, rows_hbm, cols_hbm,
             idxblk, flat, vals, outbuf, sem, *, b, n, nc, nw):
    rows_per_w = (b * n) // nw
    n_groups = rows_per_w // _G
    wid = lax.axis_index("s") * nc + lax.axis_index("c")
    lane16 = lax.iota(jnp.int32, 16)
    padmask = lane16 >= (_S - 16)

    def group_body(g, _):
        r0 = wid * rows_per_w + g * _G
        pltpu.sync_copy(idx_hbm.at[pl.ds(r0 * _SP, _G * _SP)], idxblk)

        def build_body(r, _):
            rg = r0 + r
            bb = rg // n
            ii = rg - bb * n
            rowbase = bb * n * n + ii * n
            colbase = bb * n * n + ii
            j0 = idxblk[pl.ds(r * _SP, 16)]
            j1 = idxblk[pl.ds(r * _SP + 16, 16)]
            flat[pl.ds(r * 64, 16)] = rowbase + j0
            flat[pl.ds(r * 64 + 16, 16)] = rowbase + j1
            flat[pl.ds(r * 64 + 32, 16)] = j0 * n + colbase
            flat[pl.ds(r * 64 + 48, 16)] = j1 * n + colbase
            return 0

        lax.fori_loop(0, _G, build_body, 0)

        copies = [
            pltpu.async_copy(dist_hbm.at[flat.at[pl.ds(c * 128, 128)]],
                             vals.at[pl.ds(c * 128, 128)], sem)
            for c in range(_G * 64 // 128)
        ]
        for cp in copies:
            cp.wait()

        def vsort(x):
            return plsc.sort_key_val(x, x)[0]

        def sort25(v0, v1):
            v1 = jnp.where(padmask, _POS, v1)
            a = vsort(v0)
            c = vsort(v1)
            rc = lax.rev(c, (0,))
            lo = vsort(jnp.minimum(a, rc))
            hi = vsort(jnp.maximum(a, rc))
            hi = jnp.where(padmask, 0.0, hi)
            return lo, hi

        def sort_body(r, _):
            rlo, rhi = sort25(vals[pl.ds(r * 64, 16)],
                              vals[pl.ds(r * 64 + 16, 16)])
            clo, chi = sort25(vals[pl.ds(r * 64 + 32, 16)],
                              vals[pl.ds(r * 64 + 48, 16)])
            outbuf[pl.ds(r * _SP, 16)] = rlo
            outbuf[pl.ds(r * _SP + 16, 16)] = rhi
            outbuf[pl.ds(_G * _SP + r * _SP, 16)] = clo
            outbuf[pl.ds(_G * _SP + r * _SP + 16, 16)] = chi
            return 0

        lax.fori_loop(0, _G, sort_body, 0)
        pltpu.sync_copy(outbuf.at[pl.ds(0, _G * _SP)],
                        rows_hbm.at[pl.ds(r0 * _SP, _G * _SP)])
        pltpu.sync_copy(outbuf.at[pl.ds(_G * _SP, _G * _SP)],
                        cols_hbm.at[pl.ds(r0 * _SP, _G * _SP)])
        return 0

    lax.fori_loop(0, n_groups, group_body, 0)


def _sc_gather_sort(dist_flat, idx_flat, b, n):
    info = plsc.get_sparse_core_info()
    nc, ns = info.num_cores, info.num_subcores
    nw = nc * ns
    mesh = plsc.VectorSubcoreMesh(core_axis_name="c", subcore_axis_name="s")
    kern = functools.partial(
        pl.kernel,
        mesh=mesh,
        compiler_params=pltpu.CompilerParams(needs_layout_passes=False),
        out_type=[jax.ShapeDtypeStruct((b * n * _SP,), jnp.float32)] * 2,
        scratch_types=[
            pltpu.VMEM((_G * _SP,), jnp.int32),
            pltpu.VMEM((_G * 64,), jnp.int32),
            pltpu.VMEM((_G * 64,), jnp.float32),
            pltpu.VMEM((2 * _G * _SP,), jnp.float32),
            pltpu.SemaphoreType.DMA,
        ],
    )(functools.partial(_sc_body, b=b, n=n, nc=nc, nw=nw))
    return kern(dist_flat, idx_flat)


def _stage2_body(locs_ref, rows_ref, cols_ref, iwt_ref, rwt_ref, cwt_ref,
                 g1c_r_ref, g1d_r_ref, g1c_c_ref, g1d_c_ref, aux_ref,
                 b128_ref, outr_ref, outc_ref):
    f32 = jnp.float32
    aux = aux_ref[...]
    b128 = b128_ref[...]
    e = (jnp.dot(locs_ref[0], iwt_ref[...], preferred_element_type=f32)
         + b128[0:1, :])
    remb = (jnp.dot(rows_ref[0], rwt_ref[...], preferred_element_type=f32)
            + b128[1:2, :])
    cemb = (jnp.dot(cols_ref[0], cwt_ref[...], preferred_element_type=f32)
            + b128[2:3, :])

    def gate(feat, w1c, w1d, brow, wrow, b2row):
        h = jax.nn.relu(
            jnp.dot(e, w1c, preferred_element_type=f32)
            + jnp.dot(feat, w1d, preferred_element_type=f32)
            + aux[brow:brow + 1, :])
        gp = (jnp.sum(h * aux[wrow:wrow + 1, :], axis=1, keepdims=True)
              + aux[b2row:b2row + 1, 0:1])
        g = jax.nn.sigmoid(gp)
        return g * e + (1.0 - g) * feat

    outr_ref[0] = gate(remb, g1c_r_ref[...], g1d_r_ref[...], 0, 2, 4)
    outc_ref[0] = gate(cemb, g1c_c_ref[...], g1d_c_ref[...], 1, 3, 5)


def kernel(locs, distance_matrix, params):
    b, n, _ = locs.shape
    f32 = jnp.float32

    w = _w_const(b, n)

    rows_per_blk = _R if n % _R == 0 else n
    grid1 = (b, n // rows_per_blk)
    big_spec = pl.BlockSpec((1, rows_per_blk, n), lambda i, j: (i, j, 0))
    idx = pl.pallas_call(
        functools.partial(_stage1_body, rows_per_blk=rows_per_blk, n=n),
        grid=grid1,
        in_specs=[big_spec, big_spec],
        out_specs=pl.BlockSpec((1, rows_per_blk, _SP), lambda i, j: (i, j, 0)),
        out_shape=jax.ShapeDtypeStruct((b, n, _SP), jnp.int32),
    )(distance_matrix, w)

    rows_flat, cols_flat = _sc_gather_sort(
        distance_matrix.reshape(-1), idx.reshape(-1), b, n)
    rows_sorted = rows_flat.reshape(b, n, _SP)
    cols_sorted = cols_flat.reshape(b, n, _SP)

    locs_pad = jnp.pad(locs, ((0, 0), (0, 0), (0, 6)))
    iwt = jnp.pad(params['init_W'].T, ((0, 6), (0, 0)))
    rwt = jnp.pad(params['row_W'].T, ((0, _SP - _S), (0, 0)))
    cwt = jnp.pad(params['col_W'].T, ((0, _SP - _S), (0, 0)))
    g1_r = params['grow_W1'].T
    g1_c = params['gcol_W1'].T
    ed = g1_r.shape[0] // 2
    aux = jnp.zeros((8, 2 * ed), f32)
    aux = aux.at[0, :].set(params['grow_b1'])
    aux = aux.at[1, :].set(params['gcol_b1'])
    aux = aux.at[2, :].set(params['grow_W2'][0])
    aux = aux.at[3, :].set(params['gcol_W2'][0])
    aux = aux.at[4, :].set(params['grow_b2'][0])
    aux = aux.at[5, :].set(params['gcol_b2'][0])
    b128 = jnp.zeros((8, ed), f32)
    b128 = b128.at[0, :].set(params['init_b'])
    b128 = b128.at[1, :].set(params['row_b'])
    b128 = b128.at[2, :].set(params['col_b'])

    def wspec(shape):
        return pl.BlockSpec(shape, lambda i: (0,) * len(shape))

    outr, outc = pl.pallas_call(
        _stage2_body,
        grid=(b,),
        in_specs=[
            pl.BlockSpec((1, n, 8), lambda i: (i, 0, 0)),
            pl.BlockSpec((1, n, _SP), lambda i: (i, 0, 0)),
            pl.BlockSpec((1, n, _SP), lambda i: (i, 0, 0)),
            wspec((8, ed)), wspec((_SP, ed)), wspec((_SP, ed)),
            wspec((ed, 2 * ed)), wspec((ed, 2 * ed)),
            wspec((ed, 2 * ed)), wspec((ed, 2 * ed)),
            wspec((8, 2 * ed)), wspec((8, ed)),
        ],
        out_specs=[pl.BlockSpec((1, n, ed), lambda i: (i, 0, 0))] * 2,
        out_shape=[jax.ShapeDtypeStruct((b, n, ed), f32)] * 2,
    )(locs_pad, rows_sorted, cols_sorted, iwt, rwt, cwt,
      g1_r[:ed], g1_r[ed:], g1_c[:ed], g1_c[ed:], aux, b128)

    return (outr, outc, distance_matrix)


# speedup vs baseline: 1.2845x; 1.0603x over previous
import functools

import jax
import jax.numpy as jnp
from jax import lax
from jax.experimental import pallas as pl
from jax.experimental.pallas import tpu as pltpu
from jax.experimental.pallas import tpu_sc as plsc

_S = 25
_SP = 32
_R = 512
_NEG = -1.0
_POS = 1e30
_G = 64

_W_CACHE = {}


def _w_const(b, n):
    if (b, n) not in _W_CACHE:
        u = jax.random.uniform(jax.random.key(42), (b, n, n),
                               dtype=jnp.float32, minval=1e-10, maxval=1.0)
        _W_CACHE[(b, n)] = 1.0 / (-jnp.log(u))
    return _W_CACHE[(b, n)]


def _stage1_body(d_ref, w_ref, idx_ref, *, rows_per_blk, n):
    row_base = pl.program_id(1) * rows_per_blk
    d = d_ref[0]
    w = w_ref[0]
    col_ids = jax.lax.broadcasted_iota(jnp.int32, (rows_per_blk, n), 1)
    row_ids = row_base + jax.lax.broadcasted_iota(
        jnp.int32, (rows_per_blk, n), 0)
    dproc = jnp.where(col_ids == row_ids, 1e6, d)
    s = w / (dproc + 1e-6)

    lane = jax.lax.broadcasted_iota(jnp.int32, (rows_per_blk, _SP), 1)

    def topk_body(k, carry):
        s, iacc = carry
        m = jnp.max(s, axis=1, keepdims=True)
        cand = jnp.where(s >= m, col_ids, n)
        jm = jnp.min(cand, axis=1, keepdims=True)
        iacc = jnp.where(lane == k, jm, iacc)
        s = jnp.where(cand == jm, _NEG, s)
        return s, iacc

    iacc0 = jnp.zeros((rows_per_blk, _SP), jnp.int32)
    _, iacc = jax.lax.fori_loop(0, _S, topk_body, (s, iacc0))
    idx_ref[0] = iacc


def _sc_body(dist_hbm, idx_hbm, rows_hbm, cols_hbm,
             idxblk, flat, vals, outbuf, sem, *, b, n, nc, nw):
    rows_per_w = (b * n) // nw
    n_groups = rows_per_w // _G
    wid = lax.axis_index("s") * nc + lax.axis_index("c")
    lane16 = lax.iota(jnp.int32, 16)
    padmask = lane16 >= (_S - 16)

    def group_body(g, _):
        r0 = wid * rows_per_w + g * _G
        pltpu.sync_copy(idx_hbm.at[pl.ds(r0 * _SP, _G * _SP)], idxblk)

        def build_body(r, _):
            rg = r0 + r
            bb = rg // n
            ii = rg - bb * n
            rowbase = bb * n * n + ii * n
            colbase = bb * n * n + ii
            j0 = idxblk[pl.ds(r * _SP, 16)]
            j1 = idxblk[pl.ds(r * _SP + 16, 16)]
            flat[pl.ds(r * 64, 16)] = rowbase + j0
            flat[pl.ds(r * 64 + 16, 16)] = rowbase + j1
            flat[pl.ds(r * 64 + 32, 16)] = j0 * n + colbase
            flat[pl.ds(r * 64 + 48, 16)] = j1 * n + colbase
            return 0

        lax.fori_loop(0, _G, build_body, 0)

        copies = [
            pltpu.async_copy(dist_hbm.at[flat.at[pl.ds(c * 128, 128)]],
                             vals.at[pl.ds(c * 128, 128)], sem)
            for c in range(_G * 64 // 128)
        ]
        for cp in copies:
            cp.wait()

        def vsort(x):
            return plsc.sort_key_val(x, x)[0]

        def sort25(v0, v1):
            v1 = jnp.where(padmask, _POS, v1)
            a = vsort(v0)
            c = vsort(v1)
            rc = lax.rev(c, (0,))
            lo = vsort(jnp.minimum(a, rc))
            hi = vsort(jnp.maximum(a, rc))
            hi = jnp.where(padmask, 0.0, hi)
            return lo, hi

        def sort_body(r, _):
            rlo, rhi = sort25(vals[pl.ds(r * 64, 16)],
                              vals[pl.ds(r * 64 + 16, 16)])
            clo, chi = sort25(vals[pl.ds(r * 64 + 32, 16)],
                              vals[pl.ds(r * 64 + 48, 16)])
            outbuf[pl.ds(r * _SP, 16)] = rlo
            outbuf[pl.ds(r * _SP + 16, 16)] = rhi
            outbuf[pl.ds(_G * _SP + r * _SP, 16)] = clo
            outbuf[pl.ds(_G * _SP + r * _SP + 16, 16)] = chi
            return 0

        lax.fori_loop(0, _G, sort_body, 0)
        pltpu.sync_copy(outbuf.at[pl.ds(0, _G * _SP)],
                        rows_hbm.at[pl.ds(r0 * _SP, _G * _SP)])
        pltpu.sync_copy(outbuf.at[pl.ds(_G * _SP, _G * _SP)],
                        cols_hbm.at[pl.ds(r0 * _SP, _G * _SP)])
        return 0

    lax.fori_loop(0, n_groups, group_body, 0)


def _sc_gather_sort(dist_flat, idx_flat, b, n):
    info = plsc.get_sparse_core_info()
    nc, ns = info.num_cores, info.num_subcores
    nw = nc * ns
    mesh = plsc.VectorSubcoreMesh(core_axis_name="c", subcore_axis_name="s")
    kern = functools.partial(
        pl.kernel,
        mesh=mesh,
        compiler_params=pltpu.CompilerParams(needs_layout_passes=False),
        out_type=[jax.ShapeDtypeStruct((b * n * _SP,), jnp.float32)] * 2,
        scratch_types=[
            pltpu.VMEM((_G * _SP,), jnp.int32),
            pltpu.VMEM((_G * 64,), jnp.int32),
            pltpu.VMEM((_G * 64,), jnp.float32),
            pltpu.VMEM((2 * _G * _SP,), jnp.float32),
            pltpu.SemaphoreType.DMA,
        ],
    )(functools.partial(_sc_body, b=b, n=n, nc=nc, nw=nw))
    return kern(dist_flat, idx_flat)


def _stage2_body(locs_ref, rows_ref, cols_ref, iwt_ref, rwt_ref, cwt_ref,
                 g1c_r_ref, g1d_r_ref, g1c_c_ref, g1d_c_ref, aux_ref,
                 b128_ref, outr_ref, outc_ref):
    f32 = jnp.float32
    aux = aux_ref[...]
    b128 = b128_ref[...]
    e = (jnp.dot(locs_ref[0], iwt_ref[...], preferred_element_type=f32)
         + b128[0:1, :])
    remb = (jnp.dot(rows_ref[0], rwt_ref[...], preferred_element_type=f32)
            + b128[1:2, :])
    cemb = (jnp.dot(cols_ref[0], cwt_ref[...], preferred_element_type=f32)
            + b128[2:3, :])

    def gate(feat, w1c, w1d, brow, wrow, b2row):
        h = jax.nn.relu(
            jnp.dot(e, w1c, preferred_element_type=f32)
            + jnp.dot(feat, w1d, preferred_element_type=f32)
            + aux[brow:brow + 1, :])
        gp = (jnp.sum(h * aux[wrow:wrow + 1, :], axis=1, keepdims=True)
              + aux[b2row:b2row + 1, 0:1])
        g = jax.nn.sigmoid(gp)
        return g * e + (1.0 - g) * feat

    outr_ref[0] = gate(remb, g1c_r_ref[...], g1d_r_ref[...], 0, 2, 4)
    outc_ref[0] = gate(cemb, g1c_c_ref[...], g1d_c_ref[...], 1, 3, 5)


def kernel(locs, distance_matrix, params):
    b, n, _ = locs.shape
    f32 = jnp.float32

    w = _w_const(b, n)

    rows_per_blk = _R if n % _R == 0 else n
    grid1 = (b, n // rows_per_blk)
    big_spec = pl.BlockSpec((1, rows_per_blk, n), lambda i, j: (i, j, 0))
    idx = pl.pallas_call(
        functools.partial(_stage1_body, rows_per_blk=rows_per_blk, n=n),
        grid=grid1,
        in_specs=[big_spec, big_spec],
        out_specs=pl.BlockSpec((1, rows_per_blk, _SP), lambda i, j: (i, j, 0)),
        out_shape=jax.ShapeDtypeStruct((b, n, _SP), jnp.int32),
    )(distance_matrix, w)

    rows_flat, cols_flat = _sc_gather_sort(
        distance_matrix.reshape(-1), idx.reshape(-1), b, n)
    rows_sorted = rows_flat.reshape(b, n, _SP)
    cols_sorted = cols_flat.reshape(b, n, _SP)

    locs_pad = jnp.pad(locs, ((0, 0), (0, 0), (0, 6)))
    iwt = jnp.pad(params['init_W'].T, ((0, 6), (0, 0)))
    rwt = jnp.pad(params['row_W'].T, ((0, _SP - _S), (0, 0)))
    cwt = jnp.pad(params['col_W'].T, ((0, _SP - _S), (0, 0)))
    g1_r = params['grow_W1'].T
    g1_c = params['gcol_W1'].T
    ed = g1_r.shape[0] // 2
    aux = jnp.zeros((8, 2 * ed), f32)
    aux = aux.at[0, :].set(params['grow_b1'])
    aux = aux.at[1, :].set(params['gcol_b1'])
    aux = aux.at[2, :].set(params['grow_W2'][0])
    aux = aux.at[3, :].set(params['gcol_W2'][0])
    aux = aux.at[4, :].set(params['grow_b2'][0])
    aux = aux.at[5, :].set(params['gcol_b2'][0])
    b128 = jnp.zeros((8, ed), f32)
    b128 = b128.at[0, :].set(params['init_b'])
    b128 = b128.at[1, :].set(params['row_b'])
    b128 = b128.at[2, :].set(params['col_b'])

    def wspec(shape):
        return pl.BlockSpec(shape, lambda i: (0,) * len(shape))

    outr, outc = pl.pallas_call(
        _stage2_body,
        grid=(b,),
        in_specs=[
            pl.BlockSpec((1, n, 8), lambda i: (i, 0, 0)),
            pl.BlockSpec((1, n, _SP), lambda i: (i, 0, 0)),
            pl.BlockSpec((1, n, _SP), lambda i: (i, 0, 0)),
            wspec((8, ed)), wspec((_SP, ed)), wspec((_SP, ed)),
            wspec((ed, 2 * ed)), wspec((ed, 2 * ed)),
            wspec((ed, 2 * ed)), wspec((ed, 2 * ed)),
            wspec((8, 2 * ed)), wspec((8, ed)),
        ],
        out_specs=[pl.BlockSpec((1, n, ed), lambda i: (i, 0, 0))] * 2,
        out_shape=[jax.ShapeDtypeStruct((b, n, ed), f32)] * 2,
    )(locs_pad, rows_sorted, cols_sorted, iwt, rwt, cwt,
      g1_r[:ed], g1_r[ed:], g1_c[:ed], g1_c[ed:], aux, b128)

    return (outr, outc, distance_matrix)


# speedup vs baseline: 1.3227x; 1.0297x over previous
import functools

import jax
import jax.numpy as jnp
from jax import lax
from jax.experimental import pallas as pl
from jax.experimental.pallas import tpu as pltpu
from jax.experimental.pallas import tpu_sc as plsc

_S = 25
_SP = 32
_R = 1024
_NEG = -1.0
_POS = 1e30
_G = 64

_W_CACHE = {}


def _w_const(b, n):
    if (b, n) not in _W_CACHE:
        u = jax.random.uniform(jax.random.key(42), (b, n, n),
                               dtype=jnp.float32, minval=1e-10, maxval=1.0)
        _W_CACHE[(b, n)] = 1.0 / (-jnp.log(u))
    return _W_CACHE[(b, n)]


def _stage1_body(d_ref, w_ref, idx_ref, *, rows_per_blk, n):
    row_base = pl.program_id(1) * rows_per_blk
    d = d_ref[0]
    w = w_ref[0]
    col_ids = jax.lax.broadcasted_iota(jnp.int32, (rows_per_blk, n), 1)
    row_ids = row_base + jax.lax.broadcasted_iota(
        jnp.int32, (rows_per_blk, n), 0)
    dproc = jnp.where(col_ids == row_ids, 1e6, d)
    s = w / (dproc + 1e-6)

    lane = jax.lax.broadcasted_iota(jnp.int32, (rows_per_blk, _SP), 1)

    def topk_body(k, carry):
        s, iacc = carry
        m = jnp.max(s, axis=1, keepdims=True)
        cand = jnp.where(s >= m, col_ids, n)
        jm = jnp.min(cand, axis=1, keepdims=True)
        iacc = jnp.where(lane == k, jm, iacc)
        s = jnp.where(cand == jm, _NEG, s)
        return s, iacc

    iacc0 = jnp.zeros((rows_per_blk, _SP), jnp.int32)
    _, iacc = jax.lax.fori_loop(0, _S, topk_body, (s, iacc0))
    idx_ref[0] = iacc


def _sc_body(dist_hbm, idx_hbm, rows_hbm, cols_hbm,
             idxblk, flat, vals, outbuf, sem, *, b, n, nc, nw):
    rows_per_w = (b * n) // nw
    n_groups = rows_per_w // _G
    wid = lax.axis_index("s") * nc + lax.axis_index("c")
    lane16 = lax.iota(jnp.int32, 16)
    padmask = lane16 >= (_S - 16)

    def group_body(g, _):
        r0 = wid * rows_per_w + g * _G
        pltpu.sync_copy(idx_hbm.at[pl.ds(r0 * _SP, _G * _SP)], idxblk)

        def build_body(r, _):
            rg = r0 + r
            bb = rg // n
            ii = rg - bb * n
            rowbase = bb * n * n + ii * n
            colbase = bb * n * n + ii
            j0 = idxblk[pl.ds(r * _SP, 16)]
            j1 = idxblk[pl.ds(r * _SP + 16, 16)]
            flat[pl.ds(r * 64, 16)] = rowbase + j0
            flat[pl.ds(r * 64 + 16, 16)] = rowbase + j1
            flat[pl.ds(r * 64 + 32, 16)] = j0 * n + colbase
            flat[pl.ds(r * 64 + 48, 16)] = j1 * n + colbase
            return 0

        lax.fori_loop(0, _G, build_body, 0)

        copies = [
            pltpu.async_copy(dist_hbm.at[flat.at[pl.ds(c * 128, 128)]],
                             vals.at[pl.ds(c * 128, 128)], sem)
            for c in range(_G * 64 // 128)
        ]
        for cp in copies:
            cp.wait()

        def vsort(x):
            return plsc.sort_key_val(x, x)[0]

        def sort25(v0, v1):
            v1 = jnp.where(padmask, _POS, v1)
            a = vsort(v0)
            c = vsort(v1)
            rc = lax.rev(c, (0,))
            lo = vsort(jnp.minimum(a, rc))
            hi = vsort(jnp.maximum(a, rc))
            hi = jnp.where(padmask, 0.0, hi)
            return lo, hi

        def sort_body(r, _):
            rlo, rhi = sort25(vals[pl.ds(r * 64, 16)],
                              vals[pl.ds(r * 64 + 16, 16)])
            clo, chi = sort25(vals[pl.ds(r * 64 + 32, 16)],
                              vals[pl.ds(r * 64 + 48, 16)])
            outbuf[pl.ds(r * _SP, 16)] = rlo
            outbuf[pl.ds(r * _SP + 16, 16)] = rhi
            outbuf[pl.ds(_G * _SP + r * _SP, 16)] = clo
            outbuf[pl.ds(_G * _SP + r * _SP + 16, 16)] = chi
            return 0

        lax.fori_loop(0, _G, sort_body, 0)
        pltpu.sync_copy(outbuf.at[pl.ds(0, _G * _SP)],
                        rows_hbm.at[pl.ds(r0 * _SP, _G * _SP)])
        pltpu.sync_copy(outbuf.at[pl.ds(_G * _SP, _G * _SP)],
                        cols_hbm.at[pl.ds(r0 * _SP, _G * _SP)])
        return 0

    lax.fori_loop(0, n_groups, group_body, 0)


def _sc_gather_sort(dist_flat, idx_flat, b, n):
    info = plsc.get_sparse_core_info()
    nc, ns = info.num_cores, info.num_subcores
    nw = nc * ns
    mesh = plsc.VectorSubcoreMesh(core_axis_name="c", subcore_axis_name="s")
    kern = functools.partial(
        pl.kernel,
        mesh=mesh,
        compiler_params=pltpu.CompilerParams(needs_layout_passes=False),
        out_type=[jax.ShapeDtypeStruct((b * n * _SP,), jnp.float32)] * 2,
        scratch_types=[
            pltpu.VMEM((_G * _SP,), jnp.int32),
            pltpu.VMEM((_G * 64,), jnp.int32),
            pltpu.VMEM((_G * 64,), jnp.float32),
            pltpu.VMEM((2 * _G * _SP,), jnp.float32),
            pltpu.SemaphoreType.DMA,
        ],
    )(functools.partial(_sc_body, b=b, n=n, nc=nc, nw=nw))
    return kern(dist_flat, idx_flat)


def _stage2_body(locs_ref, rows_ref, cols_ref, iwt_ref, rwt_ref, cwt_ref,
                 g1c_r_ref, g1d_r_ref, g1c_c_ref, g1d_c_ref, aux_ref,
                 b128_ref, outr_ref, outc_ref):
    f32 = jnp.float32
    aux = aux_ref[...]
    b128 = b128_ref[...]
    e = (jnp.dot(locs_ref[0], iwt_ref[...], preferred_element_type=f32)
         + b128[0:1, :])
    remb = (jnp.dot(rows_ref[0], rwt_ref[...], preferred_element_type=f32)
            + b128[1:2, :])
    cemb = (jnp.dot(cols_ref[0], cwt_ref[...], preferred_element_type=f32)
            + b128[2:3, :])

    def gate(feat, w1c, w1d, brow, wrow, b2row):
        h = jax.nn.relu(
            jnp.dot(e, w1c, preferred_element_type=f32)
            + jnp.dot(feat, w1d, preferred_element_type=f32)
            + aux[brow:brow + 1, :])
        gp = (jnp.sum(h * aux[wrow:wrow + 1, :], axis=1, keepdims=True)
              + aux[b2row:b2row + 1, 0:1])
        g = jax.nn.sigmoid(gp)
        return g * e + (1.0 - g) * feat

    outr_ref[0] = gate(remb, g1c_r_ref[...], g1d_r_ref[...], 0, 2, 4)
    outc_ref[0] = gate(cemb, g1c_c_ref[...], g1d_c_ref[...], 1, 3, 5)


def kernel(locs, distance_matrix, params):
    b, n, _ = locs.shape
    f32 = jnp.float32

    w = _w_const(b, n)

    rows_per_blk = _R if n % _R == 0 else n
    grid1 = (b, n // rows_per_blk)
    big_spec = pl.BlockSpec((1, rows_per_blk, n), lambda i, j: (i, j, 0))
    idx = pl.pallas_call(
        functools.partial(_stage1_body, rows_per_blk=rows_per_blk, n=n),
        grid=grid1,
        in_specs=[big_spec, big_spec],
        out_specs=pl.BlockSpec((1, rows_per_blk, _SP), lambda i, j: (i, j, 0)),
        out_shape=jax.ShapeDtypeStruct((b, n, _SP), jnp.int32),
    )(distance_matrix, w)

    rows_flat, cols_flat = _sc_gather_sort(
        distance_matrix.reshape(-1), idx.reshape(-1), b, n)
    rows_sorted = rows_flat.reshape(b, n, _SP)
    cols_sorted = cols_flat.reshape(b, n, _SP)

    locs_pad = jnp.pad(locs, ((0, 0), (0, 0), (0, 6)))
    iwt = jnp.pad(params['init_W'].T, ((0, 6), (0, 0)))
    rwt = jnp.pad(params['row_W'].T, ((0, _SP - _S), (0, 0)))
    cwt = jnp.pad(params['col_W'].T, ((0, _SP - _S), (0, 0)))
    g1_r = params['grow_W1'].T
    g1_c = params['gcol_W1'].T
    ed = g1_r.shape[0] // 2
    aux = jnp.zeros((8, 2 * ed), f32)
    aux = aux.at[0, :].set(params['grow_b1'])
    aux = aux.at[1, :].set(params['gcol_b1'])
    aux = aux.at[2, :].set(params['grow_W2'][0])
    aux = aux.at[3, :].set(params['gcol_W2'][0])
    aux = aux.at[4, :].set(params['grow_b2'][0])
    aux = aux.at[5, :].set(params['gcol_b2'][0])
    b128 = jnp.zeros((8, ed), f32)
    b128 = b128.at[0, :].set(params['init_b'])
    b128 = b128.at[1, :].set(params['row_b'])
    b128 = b128.at[2, :].set(params['col_b'])

    def wspec(shape):
        return pl.BlockSpec(shape, lambda i: (0,) * len(shape))

    outr, outc = pl.pallas_call(
        _stage2_body,
        grid=(b,),
        in_specs=[
            pl.BlockSpec((1, n, 8), lambda i: (i, 0, 0)),
            pl.BlockSpec((1, n, _SP), lambda i: (i, 0, 0)),
            pl.BlockSpec((1, n, _SP), lambda i: (i, 0, 0)),
            wspec((8, ed)), wspec((_SP, ed)), wspec((_SP, ed)),
            wspec((ed, 2 * ed)), wspec((ed, 2 * ed)),
            wspec((ed, 2 * ed)), wspec((ed, 2 * ed)),
            wspec((8, 2 * ed)), wspec((8, ed)),
        ],
        out_specs=[pl.BlockSpec((1, n, ed), lambda i: (i, 0, 0))] * 2,
        out_shape=[jax.ShapeDtypeStruct((b, n, ed), f32)] * 2,
    )(locs_pad, rows_sorted, cols_sorted, iwt, rwt, cwt,
      g1_r[:ed], g1_r[ed:], g1_c[:ed], g1_c[ed:], aux, b128)

    return (outr, outc, distance_matrix)
